# baseline (speedup 1.0000x reference)
import functools

import numpy as np
import jax
import jax.numpy as jnp
from jax.experimental import pallas as pl
from jax.experimental.pallas import tpu as pltpu

_G = 128
_VMEM_LIMIT_BYTES = 48 * 1024 * 1024


def _fold_kernel(x_ref, m_ref, o_ref, *, inv_area):
    x = x_ref[...].astype(jnp.bfloat16)
    s = jnp.dot(x, m_ref[...], preferred_element_type=jnp.float32)
    o_ref[...] = s * inv_area


def kernel(x):
    n, c, h, w = x.shape
    rows = n * c
    cols = h * w
    inv_area = 1.0 / float(cols)

    x2 = x.reshape(rows, cols)
    pad_rows = (-rows) % _G
    if pad_rows:
        x2 = jnp.concatenate(
            [x2, jnp.zeros((pad_rows, cols), x2.dtype)], axis=0)
    groups = x2.shape[0] // _G
    x3 = x2.reshape(groups, _G * cols)

    src = np.arange(_G * cols, dtype=np.int32) // cols
    m_np = (src[:, None] == np.arange(_G, dtype=np.int32)[None, :])
    m = jnp.asarray(m_np.astype(np.float32), dtype=jnp.bfloat16)

    tg = min(256, groups)
    num_tiles = pl.cdiv(groups, tg)

    out = pl.pallas_call(
        functools.partial(_fold_kernel, inv_area=inv_area),
        out_shape=jax.ShapeDtypeStruct((num_tiles * tg, _G), x.dtype),
        grid=(num_tiles,),
        in_specs=[
            pl.BlockSpec((tg, _G * cols), lambda i: (i, 0)),
            pl.BlockSpec((_G * cols, _G), lambda i: (0, 0)),
        ],
        out_specs=pl.BlockSpec((tg, _G), lambda i: (i, 0)),
        compiler_params=pltpu.CompilerParams(
            dimension_semantics=("parallel",),
            vmem_limit_bytes=_VMEM_LIMIT_BYTES,
        ),
    )(x3, m)

    return out.reshape(-1)[:rows].reshape(n, c, 1, 1)


# speedup vs baseline: 30.1473x; 30.1473x over previous
import functools

import jax
import jax.numpy as jnp
from jax.experimental import pallas as pl
from jax.experimental.pallas import tpu as pltpu

_VMEM_LIMIT_BYTES = 64 * 1024 * 1024


def _pool_kernel(x_ref, o_ref, *, inv_area):
    o_ref[...] = (jnp.sum(x_ref[...], axis=0) * inv_area).astype(o_ref.dtype)


def kernel(x):
    n, c, h, w = x.shape
    hw = h * w
    inv_area = 1.0 / float(hw)

    xt = jnp.transpose(x, (2, 3, 0, 1)).reshape(hw, n, c)

    bn = n
    for cand in (16, 8, 4, 2, 1):
        if n % cand == 0:
            bn = cand
            break
    num_tiles = n // bn

    out = pl.pallas_call(
        functools.partial(_pool_kernel, inv_area=inv_area),
        out_shape=jax.ShapeDtypeStruct((n, c), x.dtype),
        grid=(num_tiles,),
        in_specs=[pl.BlockSpec((hw, bn, c), lambda i: (0, i, 0))],
        out_specs=pl.BlockSpec((bn, c), lambda i: (i, 0)),
        compiler_params=pltpu.CompilerParams(
            dimension_semantics=("parallel",),
            vmem_limit_bytes=_VMEM_LIMIT_BYTES,
        ),
    )(xt)

    return out.reshape(n, c, 1, 1)
